# Initial kernel scaffold; baseline (speedup 1.0000x reference)
import jax
import jax.numpy as jnp
from jax.experimental import pallas as pl


def kernel(x, edge_index, edge_weight, W1, b1, W2, b2, Wlin, blin):
    raise NotImplementedError("write your pallas kernel here")


# speedup vs baseline: 6.7767x; 6.7767x over previous
import functools

import jax
import jax.numpy as jnp
from jax import lax
from jax.experimental import pallas as pl
from jax.experimental.pallas import tpu as pltpu
from jax.experimental.pallas import tpu_sc as plsc

N_NODES = 10000
D_FEAT = 128
HALF = 64
NC = 2
NS = 16
CHUNK = 128
NPAD = 10240
ROWS_PER_TILE = NPAD // NS
NCHUNKS = 2560
EPAD = NCHUNKS * CHUNK
DEG_CHUNKS_PER_TILE = NCHUNKS // (NC * NS)
SPMM_CHUNKS_PER_TILE = NCHUNKS // NS

_mesh = plsc.VectorSubcoreMesh(core_axis_name="c", subcore_axis_name="s")


@functools.partial(
    pl.kernel,
    out_type=jax.ShapeDtypeStruct((NC, NPAD), jnp.float32),
    mesh=_mesh,
    scratch_types=[
        pltpu.VMEM((DEG_CHUNKS_PER_TILE, CHUNK), jnp.int32),
        pltpu.VMEM((DEG_CHUNKS_PER_TILE, CHUNK), jnp.float32),
        pltpu.VMEM((ROWS_PER_TILE,), jnp.float32),
        pltpu.VMEM_SHARED((NPAD,), jnp.float32),
    ],
    compiler_params=pltpu.CompilerParams(use_tc_tiling_on_sc=False),
)
def _deg_kernel(col2d, w2d, deg_out, col_blk, w_blk, zbuf, deg_sh):
    c = lax.axis_index("c")
    s = lax.axis_index("s")
    base = (c * NS + s) * DEG_CHUNKS_PER_TILE
    pltpu.sync_copy(col2d.at[pl.ds(base, DEG_CHUNKS_PER_TILE)], col_blk)
    pltpu.sync_copy(w2d.at[pl.ds(base, DEG_CHUNKS_PER_TILE)], w_blk)

    def _zero(i, _):
        zbuf[pl.ds(i * 16, 16)] = jnp.zeros((16,), jnp.float32)
        return 0

    lax.fori_loop(0, ROWS_PER_TILE // 16, _zero, 0)
    pltpu.sync_copy(zbuf, deg_sh.at[pl.ds(s * ROWS_PER_TILE, ROWS_PER_TILE)])
    plsc.subcore_barrier()

    def _step(j, _):
        pltpu.sync_copy(w_blk.at[j], deg_sh.at[col_blk.at[j]], add=True)
        return 0

    lax.fori_loop(0, DEG_CHUNKS_PER_TILE, _step, 0)
    plsc.subcore_barrier()
    pltpu.sync_copy(
        deg_sh.at[pl.ds(s * ROWS_PER_TILE, ROWS_PER_TILE)],
        deg_out.at[c, pl.ds(s * ROWS_PER_TILE, ROWS_PER_TILE)],
    )


@functools.partial(
    pl.kernel,
    out_type=jax.ShapeDtypeStruct((NC, NPAD, HALF), jnp.float32),
    mesh=_mesh,
    scratch_types=[
        pltpu.VMEM((SPMM_CHUNKS_PER_TILE, CHUNK), jnp.int32),
        pltpu.VMEM((SPMM_CHUNKS_PER_TILE, CHUNK), jnp.int32),
        pltpu.VMEM((SPMM_CHUNKS_PER_TILE, CHUNK), jnp.float32),
        pltpu.VMEM((CHUNK, HALF), jnp.float32),
        pltpu.VMEM((CHUNK, HALF), jnp.float32),
        pltpu.VMEM_SHARED((NPAD, HALF), jnp.float32),
    ],
    compiler_params=pltpu.CompilerParams(use_tc_tiling_on_sc=False),
)
def _spmm_kernel(t_hbm, row2d, col2d, w2d, acc_out,
                 row_blk, col_blk, w_blk, gbuf, zbuf, acc_sh):
    c = lax.axis_index("c")
    s = lax.axis_index("s")
    base = s * SPMM_CHUNKS_PER_TILE
    pltpu.sync_copy(row2d.at[pl.ds(base, SPMM_CHUNKS_PER_TILE)], row_blk)
    pltpu.sync_copy(col2d.at[pl.ds(base, SPMM_CHUNKS_PER_TILE)], col_blk)
    pltpu.sync_copy(w2d.at[pl.ds(base, SPMM_CHUNKS_PER_TILE)], w_blk)

    def _zrow(r, _):
        for k in range(HALF // 16):
            zbuf[r, pl.ds(k * 16, 16)] = jnp.zeros((16,), jnp.float32)
        return 0

    lax.fori_loop(0, CHUNK, _zrow, 0)
    for i in range(ROWS_PER_TILE // CHUNK):
        pltpu.sync_copy(
            zbuf, acc_sh.at[pl.ds(s * ROWS_PER_TILE + i * CHUNK, CHUNK)]
        )
    plsc.subcore_barrier()

    def _step(j, _):
        pltpu.sync_copy(t_hbm.at[c].at[row_blk.at[j]], gbuf)

        def _scale16(g, _2):
            wv = w_blk[j, pl.ds(g * 16, 16)]
            for e_l in range(16):
                wsc = wv[e_l]
                e = g * 16 + e_l
                for k in range(HALF // 16):
                    sl = pl.ds(k * 16, 16)
                    gbuf[e, sl] = gbuf[e, sl] * wsc
            return 0

        lax.fori_loop(0, CHUNK // 16, _scale16, 0)
        pltpu.sync_copy(gbuf, acc_sh.at[col_blk.at[j]], add=True)
        return 0

    lax.fori_loop(0, SPMM_CHUNKS_PER_TILE, _step, 0)
    plsc.subcore_barrier()
    pltpu.sync_copy(
        acc_sh.at[pl.ds(s * ROWS_PER_TILE, ROWS_PER_TILE)],
        acc_out.at[c, pl.ds(s * ROWS_PER_TILE, ROWS_PER_TILE)],
    )


def _pre_body(degp_ref, x_ref, w1_ref, t_ref, dis_ref):
    deg = degp_ref[0] + degp_ref[1] + 1.0
    dis = lax.rsqrt(deg)
    dis_ref[...] = dis
    xw = jnp.dot(x_ref[...], w1_ref[...], preferred_element_type=jnp.float32)
    t = xw * dis[:, None]
    t_ref[0] = t[:, :HALF]
    t_ref[1] = t[:, HALF:]


def _mid_body(acc_ref, t1_ref, dis_ref, b1_ref, w2_ref, t2_ref):
    pre = jnp.concatenate(
        [acc_ref[0] + t1_ref[0], acc_ref[1] + t1_ref[1]], axis=1
    )
    dis = dis_ref[...]
    h = jnp.maximum(pre * dis[:, None] + b1_ref[...][None, :], 0.0)
    xw2 = jnp.dot(h, w2_ref[...], preferred_element_type=jnp.float32)
    t2 = xw2 * dis[:, None]
    t2_ref[0] = t2[:, :HALF]
    t2_ref[1] = t2[:, HALF:]


def _post_body(acc_ref, t2_ref, dis_ref, b2_ref, wlin_ref, blin_ref, out_ref):
    pre = jnp.concatenate(
        [acc_ref[0] + t2_ref[0], acc_ref[1] + t2_ref[1]], axis=1
    )
    dis = dis_ref[...]
    h2 = jnp.maximum(pre * dis[:, None] + b2_ref[...][None, :], 0.0)
    y = jnp.dot(h2, wlin_ref[...], preferred_element_type=jnp.float32)
    out_ref[...] = y[:N_NODES, :] + blin_ref[...][None, :]


_pre_call = pl.pallas_call(
    _pre_body,
    out_shape=(
        jax.ShapeDtypeStruct((NC, NPAD, HALF), jnp.float32),
        jax.ShapeDtypeStruct((NPAD,), jnp.float32),
    ),
)

_mid_call = pl.pallas_call(
    _mid_body,
    out_shape=jax.ShapeDtypeStruct((NC, NPAD, HALF), jnp.float32),
)

_post_call = pl.pallas_call(
    _post_body,
    out_shape=jax.ShapeDtypeStruct((N_NODES, 1), jnp.float32),
)


@jax.jit
def kernel(x, edge_index, edge_weight, W1, b1, W2, b2, Wlin, blin):
    row = edge_index[0].astype(jnp.int32)
    col = edge_index[1].astype(jnp.int32)
    w = edge_weight.astype(jnp.float32)
    e = row.shape[0]
    pad = EPAD - e
    row2d = jnp.concatenate([row, jnp.zeros((pad,), jnp.int32)]).reshape(
        NCHUNKS, CHUNK
    )
    col2d = jnp.concatenate([col, jnp.zeros((pad,), jnp.int32)]).reshape(
        NCHUNKS, CHUNK
    )
    w2d = jnp.concatenate([w, jnp.zeros((pad,), jnp.float32)]).reshape(
        NCHUNKS, CHUNK
    )
    x_pad = jnp.zeros((NPAD, D_FEAT), jnp.float32).at[:N_NODES].set(x)

    degp = _deg_kernel(col2d, w2d)
    t1, dis = _pre_call(degp, x_pad, W1)
    acc1 = _spmm_kernel(t1, row2d, col2d, w2d)
    t2 = _mid_call(acc1, t1, dis, b1, W2)
    acc2 = _spmm_kernel(t2, row2d, col2d, w2d)
    return _post_call(acc2, t2, dis, b2, Wlin, blin)


# speedup vs baseline: 11.3134x; 1.6695x over previous
import functools

import jax
import jax.numpy as jnp
from jax import lax
from jax.experimental import pallas as pl
from jax.experimental.pallas import tpu as pltpu
from jax.experimental.pallas import tpu_sc as plsc

N_NODES = 10000
D_FEAT = 128
HALF = 64
NC = 2
NS = 16
CHUNK = 128
NPAD = 10240
ROWS_PER_TILE = NPAD // NS
NCHUNKS = 2560
EPAD = NCHUNKS * CHUNK
DEG_CHUNKS_PER_TILE = NCHUNKS // (NC * NS)
SPMM_CHUNKS_PER_TILE = NCHUNKS // NS
KSUP = 1

_mesh = plsc.VectorSubcoreMesh(core_axis_name="c", subcore_axis_name="s")


@functools.partial(
    pl.kernel,
    out_type=jax.ShapeDtypeStruct((NC, NPAD), jnp.float32),
    mesh=_mesh,
    scratch_types=[
        pltpu.VMEM((DEG_CHUNKS_PER_TILE, CHUNK), jnp.int32),
        pltpu.VMEM((DEG_CHUNKS_PER_TILE, CHUNK), jnp.float32),
        pltpu.VMEM((ROWS_PER_TILE,), jnp.float32),
        pltpu.VMEM_SHARED((NPAD,), jnp.float32),
    ],
    compiler_params=pltpu.CompilerParams(use_tc_tiling_on_sc=False),
)
def _deg_kernel(col2d, w2d, deg_out, col_blk, w_blk, zbuf, deg_sh):
    c = lax.axis_index("c")
    s = lax.axis_index("s")
    base = (c * NS + s) * DEG_CHUNKS_PER_TILE
    pltpu.sync_copy(col2d.at[pl.ds(base, DEG_CHUNKS_PER_TILE)], col_blk)
    pltpu.sync_copy(w2d.at[pl.ds(base, DEG_CHUNKS_PER_TILE)], w_blk)

    def _zero(i, _):
        zbuf[pl.ds(i * 16, 16)] = jnp.zeros((16,), jnp.float32)
        return 0

    lax.fori_loop(0, ROWS_PER_TILE // 16, _zero, 0)
    pltpu.sync_copy(zbuf, deg_sh.at[pl.ds(s * ROWS_PER_TILE, ROWS_PER_TILE)])
    plsc.subcore_barrier()

    def _step(j, _):
        pltpu.sync_copy(w_blk.at[j], deg_sh.at[col_blk.at[j]], add=True)
        return 0

    lax.fori_loop(0, DEG_CHUNKS_PER_TILE, _step, 0)
    plsc.subcore_barrier()
    pltpu.sync_copy(
        deg_sh.at[pl.ds(s * ROWS_PER_TILE, ROWS_PER_TILE)],
        deg_out.at[c, pl.ds(s * ROWS_PER_TILE, ROWS_PER_TILE)],
    )


@functools.partial(
    pl.kernel,
    out_type=jax.ShapeDtypeStruct((NC, NPAD, HALF), jnp.float32),
    mesh=_mesh,
    scratch_types=[
        pltpu.VMEM((SPMM_CHUNKS_PER_TILE, CHUNK), jnp.int32),
        pltpu.VMEM((SPMM_CHUNKS_PER_TILE, CHUNK), jnp.int32),
        pltpu.VMEM((SPMM_CHUNKS_PER_TILE, CHUNK), jnp.float32),
        pltpu.VMEM((CHUNK, HALF), jnp.float32),
        pltpu.VMEM((CHUNK, HALF), jnp.float32),
        pltpu.VMEM((CHUNK, HALF), jnp.float32),
        pltpu.VMEM_SHARED((NPAD, HALF), jnp.float32),
        pltpu.SemaphoreType.DMA,
        pltpu.SemaphoreType.DMA,
    ],
    compiler_params=pltpu.CompilerParams(use_tc_tiling_on_sc=False),
)
def _spmm_kernel(t_hbm, row2d, col2d, w2d, acc_out,
                 row_blk, col_blk, w_blk, gbuf, s0, s1, acc_sh, ssem0, ssem1):
    c = lax.axis_index("c")
    s = lax.axis_index("s")
    base = s * SPMM_CHUNKS_PER_TILE
    pltpu.sync_copy(row2d.at[pl.ds(base, SPMM_CHUNKS_PER_TILE)], row_blk)
    pltpu.sync_copy(col2d.at[pl.ds(base, SPMM_CHUNKS_PER_TILE)], col_blk)
    pltpu.sync_copy(w2d.at[pl.ds(base, SPMM_CHUNKS_PER_TILE)], w_blk)

    def _zrow(r, _):
        for k in range(HALF // 16):
            s0[r, pl.ds(k * 16, 16)] = jnp.zeros((16,), jnp.float32)
        return 0

    lax.fori_loop(0, CHUNK, _zrow, 0)
    for i in range(ROWS_PER_TILE // CHUNK):
        pltpu.sync_copy(
            s0, acc_sh.at[pl.ds(s * ROWS_PER_TILE + i * CHUNK, CHUNK)]
        )
    plsc.subcore_barrier()

    def _scale(j, sb):
        def _scale16(g, _2):
            wv = w_blk[j, pl.ds(g * 16, 16)]
            for e_l in range(16):
                wsc = wv[e_l]
                e = g * 16 + e_l
                for k in range(HALF // 16):
                    sl = pl.ds(k * 16, 16)
                    sb[e, sl] = gbuf[e, sl] * wsc
            return 0

        lax.fori_loop(0, CHUNK // 16, _scale16, 0)

    def _step(i, _):
        j = 2 * i
        pltpu.sync_copy(t_hbm.at[c].at[row_blk.at[j]], gbuf)
        _scale(j, s0)
        pltpu.async_copy(s0, acc_sh.at[col_blk.at[j]], ssem0, add=True)
        pltpu.sync_copy(t_hbm.at[c].at[row_blk.at[j + 1]], gbuf)
        _scale(j + 1, s1)
        pltpu.async_copy(s1, acc_sh.at[col_blk.at[j + 1]], ssem1, add=True)
        pltpu.make_async_copy(s0, acc_sh.at[col_blk.at[j]], ssem0).wait()
        pltpu.make_async_copy(s1, acc_sh.at[col_blk.at[j + 1]], ssem1).wait()
        return 0

    lax.fori_loop(0, SPMM_CHUNKS_PER_TILE // 2, _step, 0)
    plsc.subcore_barrier()
    pltpu.sync_copy(
        acc_sh.at[pl.ds(s * ROWS_PER_TILE, ROWS_PER_TILE)],
        acc_out.at[c, pl.ds(s * ROWS_PER_TILE, ROWS_PER_TILE)],
    )


def _pre_body(degp_ref, x_ref, w1_ref, t_ref, dis_ref):
    deg = degp_ref[0] + degp_ref[1] + 1.0
    dis = lax.rsqrt(deg)
    dis_ref[...] = dis
    xw = jnp.dot(x_ref[...], w1_ref[...], preferred_element_type=jnp.float32)
    t = xw * dis[:, None]
    t_ref[0] = t[:, :HALF]
    t_ref[1] = t[:, HALF:]


def _mid_body(acc_ref, t1_ref, dis_ref, b1_ref, w2_ref, t2_ref):
    pre = jnp.concatenate(
        [acc_ref[0] + t1_ref[0], acc_ref[1] + t1_ref[1]], axis=1
    )
    dis = dis_ref[...]
    h = jnp.maximum(pre * dis[:, None] + b1_ref[...][None, :], 0.0)
    xw2 = jnp.dot(h, w2_ref[...], preferred_element_type=jnp.float32)
    t2 = xw2 * dis[:, None]
    t2_ref[0] = t2[:, :HALF]
    t2_ref[1] = t2[:, HALF:]


def _post_body(acc_ref, t2_ref, dis_ref, b2_ref, wlin_ref, blin_ref, out_ref):
    pre = jnp.concatenate(
        [acc_ref[0] + t2_ref[0], acc_ref[1] + t2_ref[1]], axis=1
    )
    dis = dis_ref[...]
    h2 = jnp.maximum(pre * dis[:, None] + b2_ref[...][None, :], 0.0)
    y = jnp.dot(h2, wlin_ref[...], preferred_element_type=jnp.float32)
    out_ref[...] = y[:N_NODES, :] + blin_ref[...][None, :]


_pre_call = pl.pallas_call(
    _pre_body,
    out_shape=(
        jax.ShapeDtypeStruct((NC, NPAD, HALF), jnp.float32),
        jax.ShapeDtypeStruct((NPAD,), jnp.float32),
    ),
)

_mid_call = pl.pallas_call(
    _mid_body,
    out_shape=jax.ShapeDtypeStruct((NC, NPAD, HALF), jnp.float32),
)

_post_call = pl.pallas_call(
    _post_body,
    out_shape=jax.ShapeDtypeStruct((N_NODES, 1), jnp.float32),
)


@jax.jit
def kernel(x, edge_index, edge_weight, W1, b1, W2, b2, Wlin, blin):
    row = edge_index[0].astype(jnp.int32)
    col = edge_index[1].astype(jnp.int32)
    w = edge_weight.astype(jnp.float32)
    e = row.shape[0]
    pad = EPAD - e
    row2d = jnp.concatenate([row, jnp.zeros((pad,), jnp.int32)]).reshape(
        NCHUNKS, CHUNK
    )
    col2d = jnp.concatenate([col, jnp.zeros((pad,), jnp.int32)]).reshape(
        NCHUNKS, CHUNK
    )
    w2d = jnp.concatenate([w, jnp.zeros((pad,), jnp.float32)]).reshape(
        NCHUNKS, CHUNK
    )
    x_pad = jnp.zeros((NPAD, D_FEAT), jnp.float32).at[:N_NODES].set(x)


    degp = _deg_kernel(col2d, w2d)
    t1, dis = _pre_call(degp, x_pad, W1)
    acc1 = _spmm_kernel(t1, row2d, col2d, w2d)
    t2 = _mid_call(acc1, t1, dis, b1, W2)
    acc2 = _spmm_kernel(t2, row2d, col2d, w2d)
    return _post_call(acc2, t2, dis, b2, Wlin, blin)


# speedup vs baseline: 16.3150x; 1.4421x over previous
import functools

import jax
import jax.numpy as jnp
from jax import lax
from jax.experimental import pallas as pl
from jax.experimental.pallas import tpu as pltpu
from jax.experimental.pallas import tpu_sc as plsc

N_NODES = 10000
D_FEAT = 128
HALF = 64
QW = 32
NC = 2
NS = 16
CHUNK = 128
NPAD = 10240
ROWS_PER_TILE = NPAD // NS
RBLK = 2560
NCHUNKS = 2560
EPAD = NCHUNKS * CHUNK
DEG_CHUNKS_PER_TILE = NCHUNKS // (NC * NS)
SPMM_CHUNKS_PER_TILE = NCHUNKS // NS
KSUP = 1

_mesh = plsc.VectorSubcoreMesh(core_axis_name="c", subcore_axis_name="s")


@functools.partial(
    pl.kernel,
    out_type=jax.ShapeDtypeStruct((NC, NPAD), jnp.float32),
    mesh=_mesh,
    scratch_types=[
        pltpu.VMEM((DEG_CHUNKS_PER_TILE, CHUNK), jnp.int32),
        pltpu.VMEM((DEG_CHUNKS_PER_TILE, CHUNK), jnp.float32),
        pltpu.VMEM((ROWS_PER_TILE,), jnp.float32),
        pltpu.VMEM_SHARED((NPAD,), jnp.float32),
    ],
    compiler_params=pltpu.CompilerParams(use_tc_tiling_on_sc=False),
)
def _deg_kernel(col2d, w2d, deg_out, col_blk, w_blk, zbuf, deg_sh):
    c = lax.axis_index("c")
    s = lax.axis_index("s")
    base = (c * NS + s) * DEG_CHUNKS_PER_TILE
    pltpu.sync_copy(col2d.at[pl.ds(base, DEG_CHUNKS_PER_TILE)], col_blk)
    pltpu.sync_copy(w2d.at[pl.ds(base, DEG_CHUNKS_PER_TILE)], w_blk)

    def _zero(i, _):
        zbuf[pl.ds(i * 16, 16)] = jnp.zeros((16,), jnp.float32)
        return 0

    lax.fori_loop(0, ROWS_PER_TILE // 16, _zero, 0)
    pltpu.sync_copy(zbuf, deg_sh.at[pl.ds(s * ROWS_PER_TILE, ROWS_PER_TILE)])
    plsc.subcore_barrier()

    def _step(j, _):
        pltpu.sync_copy(w_blk.at[j], deg_sh.at[col_blk.at[j]], add=True)
        return 0

    lax.fori_loop(0, DEG_CHUNKS_PER_TILE, _step, 0)
    plsc.subcore_barrier()
    pltpu.sync_copy(
        deg_sh.at[pl.ds(s * ROWS_PER_TILE, ROWS_PER_TILE)],
        deg_out.at[c, pl.ds(s * ROWS_PER_TILE, ROWS_PER_TILE)],
    )


@functools.partial(
    pl.kernel,
    out_type=jax.ShapeDtypeStruct((4, NPAD, QW), jnp.float32),
    mesh=_mesh,
    scratch_types=[
        pltpu.VMEM((SPMM_CHUNKS_PER_TILE, CHUNK), jnp.int32),
        pltpu.VMEM((SPMM_CHUNKS_PER_TILE, CHUNK), jnp.int32),
        pltpu.VMEM((SPMM_CHUNKS_PER_TILE, CHUNK), jnp.float32),
        pltpu.VMEM((CHUNK, QW), jnp.float32),
        pltpu.VMEM((CHUNK, QW), jnp.float32),
        pltpu.VMEM((CHUNK, QW), jnp.float32),
        pltpu.VMEM_SHARED((NPAD, QW), jnp.float32),
        pltpu.VMEM_SHARED((NPAD, QW), jnp.float32),
        pltpu.SemaphoreType.DMA,
        pltpu.SemaphoreType.DMA,
    ],
    compiler_params=pltpu.CompilerParams(use_tc_tiling_on_sc=False),
)
def _spmm_kernel(t_hbm, row2d, col2d, w2d, acc_out,
                 row_blk, col_blk, w_blk, gbuf, s0, s1, table_q, acc_q,
                 ssem0, ssem1):
    c = lax.axis_index("c")
    s = lax.axis_index("s")
    base = s * SPMM_CHUNKS_PER_TILE
    pltpu.sync_copy(row2d.at[pl.ds(base, SPMM_CHUNKS_PER_TILE)], row_blk)
    pltpu.sync_copy(col2d.at[pl.ds(base, SPMM_CHUNKS_PER_TILE)], col_blk)
    pltpu.sync_copy(w2d.at[pl.ds(base, SPMM_CHUNKS_PER_TILE)], w_blk)

    def _scale(j, sb):
        def _scale16(g, _2):
            wv = w_blk[j, pl.ds(g * 16, 16)]
            for e_l in range(16):
                wsc = wv[e_l]
                e = g * 16 + e_l
                for k in range(QW // 16):
                    sl = pl.ds(k * 16, 16)
                    sb[e, sl] = gbuf[e, sl] * wsc
            return 0

        lax.fori_loop(0, CHUNK // 16, _scale16, 0)

    for fq in range(2):
        qg = c * 2 + fq
        pltpu.sync_copy(
            t_hbm.at[qg, pl.ds(s * ROWS_PER_TILE, ROWS_PER_TILE)],
            table_q.at[pl.ds(s * ROWS_PER_TILE, ROWS_PER_TILE)],
        )

        def _zrow(r, _):
            for k in range(QW // 16):
                s0[r, pl.ds(k * 16, 16)] = jnp.zeros((16,), jnp.float32)
            return 0

        lax.fori_loop(0, CHUNK, _zrow, 0)
        for i in range(ROWS_PER_TILE // CHUNK):
            pltpu.sync_copy(
                s0, acc_q.at[pl.ds(s * ROWS_PER_TILE + i * CHUNK, CHUNK)]
            )
        plsc.subcore_barrier()

        def _step(i, _):
            j = 2 * i
            pltpu.sync_copy(table_q.at[row_blk.at[j]], gbuf)
            _scale(j, s0)
            pltpu.async_copy(s0, acc_q.at[col_blk.at[j]], ssem0, add=True)
            pltpu.sync_copy(table_q.at[row_blk.at[j + 1]], gbuf)
            _scale(j + 1, s1)
            pltpu.async_copy(s1, acc_q.at[col_blk.at[j + 1]], ssem1, add=True)
            pltpu.make_async_copy(s0, acc_q.at[col_blk.at[j]], ssem0).wait()
            pltpu.make_async_copy(s1, acc_q.at[col_blk.at[j + 1]], ssem1).wait()
            return 0

        lax.fori_loop(0, SPMM_CHUNKS_PER_TILE // 2, _step, 0)
        plsc.subcore_barrier()
        pltpu.sync_copy(
            acc_q.at[pl.ds(s * ROWS_PER_TILE, ROWS_PER_TILE)],
            acc_out.at[qg, pl.ds(s * ROWS_PER_TILE, ROWS_PER_TILE)],
        )
        plsc.subcore_barrier()


def _pre_body(degp_ref, x_ref, w1_ref, t_ref, dis_ref):
    deg = degp_ref[0] + degp_ref[1] + 1.0
    dis = lax.rsqrt(deg)
    dis_ref[...] = dis
    sl = pl.ds(pl.program_id(0) * RBLK, RBLK)
    disb = lax.rsqrt(degp_ref[0, sl] + degp_ref[1, sl] + 1.0)
    xw = jnp.dot(x_ref[...], w1_ref[...], preferred_element_type=jnp.float32)
    t = xw * disb[:, None]
    for q in range(4):
        t_ref[q] = t[:, q * QW:(q + 1) * QW]


def _mid_body(acc_ref, t1_ref, dis_ref, b1_ref, w2_ref, t2_ref):
    pre = jnp.concatenate(
        [acc_ref[q] + t1_ref[q] for q in range(4)], axis=1
    )
    dis = dis_ref[pl.ds(pl.program_id(0) * RBLK, RBLK)]
    h = jnp.maximum(pre * dis[:, None] + b1_ref[...][None, :], 0.0)
    xw2 = jnp.dot(h, w2_ref[...], preferred_element_type=jnp.float32)
    t2 = xw2 * dis[:, None]
    for q in range(4):
        t2_ref[q] = t2[:, q * QW:(q + 1) * QW]


def _post_body(acc_ref, t2_ref, dis_ref, b2_ref, wlin_ref, blin_ref, out_ref):
    pre = jnp.concatenate(
        [acc_ref[q] + t2_ref[q] for q in range(4)], axis=1
    )
    dis = dis_ref[pl.ds(pl.program_id(0) * RBLK, RBLK)]
    h2 = jnp.maximum(pre * dis[:, None] + b2_ref[...][None, :], 0.0)
    y = jnp.dot(h2, wlin_ref[...], preferred_element_type=jnp.float32)
    out_ref[...] = y + blin_ref[...][None, :]


_pre_call = pl.pallas_call(
    _pre_body,
    grid=(NPAD // RBLK,),
    in_specs=[
        pl.BlockSpec((NC, NPAD), lambda i: (0, 0)),
        pl.BlockSpec((RBLK, D_FEAT), lambda i: (i, 0)),
        pl.BlockSpec((D_FEAT, D_FEAT), lambda i: (0, 0)),
    ],
    out_specs=(
        pl.BlockSpec((4, RBLK, QW), lambda i: (0, i, 0)),
        pl.BlockSpec((NPAD,), lambda i: (0,)),
    ),
    out_shape=(
        jax.ShapeDtypeStruct((4, NPAD, QW), jnp.float32),
        jax.ShapeDtypeStruct((NPAD,), jnp.float32),
    ),
)

_mid_call = pl.pallas_call(
    _mid_body,
    grid=(NPAD // RBLK,),
    in_specs=[
        pl.BlockSpec((4, RBLK, QW), lambda i: (0, i, 0)),
        pl.BlockSpec((4, RBLK, QW), lambda i: (0, i, 0)),
        pl.BlockSpec((NPAD,), lambda i: (0,)),
        pl.BlockSpec((D_FEAT,), lambda i: (0,)),
        pl.BlockSpec((D_FEAT, D_FEAT), lambda i: (0, 0)),
    ],
    out_specs=pl.BlockSpec((4, RBLK, QW), lambda i: (0, i, 0)),
    out_shape=jax.ShapeDtypeStruct((4, NPAD, QW), jnp.float32),
)

_post_call = pl.pallas_call(
    _post_body,
    grid=(NPAD // RBLK,),
    in_specs=[
        pl.BlockSpec((4, RBLK, QW), lambda i: (0, i, 0)),
        pl.BlockSpec((4, RBLK, QW), lambda i: (0, i, 0)),
        pl.BlockSpec((NPAD,), lambda i: (0,)),
        pl.BlockSpec((D_FEAT,), lambda i: (0,)),
        pl.BlockSpec((D_FEAT, 1), lambda i: (0, 0)),
        pl.BlockSpec((1,), lambda i: (0,)),
    ],
    out_specs=pl.BlockSpec((RBLK, 1), lambda i: (i, 0)),
    out_shape=jax.ShapeDtypeStruct((N_NODES, 1), jnp.float32),
)


@jax.jit
def kernel(x, edge_index, edge_weight, W1, b1, W2, b2, Wlin, blin):
    row = edge_index[0].astype(jnp.int32)
    col = edge_index[1].astype(jnp.int32)
    w = edge_weight.astype(jnp.float32)
    e = row.shape[0]
    pad = EPAD - e
    row2d = jnp.concatenate([row, jnp.zeros((pad,), jnp.int32)]).reshape(
        NCHUNKS, CHUNK
    )
    col2d = jnp.concatenate([col, jnp.zeros((pad,), jnp.int32)]).reshape(
        NCHUNKS, CHUNK
    )
    w2d = jnp.concatenate([w, jnp.zeros((pad,), jnp.float32)]).reshape(
        NCHUNKS, CHUNK
    )
    x_pad = jnp.zeros((NPAD, D_FEAT), jnp.float32).at[:N_NODES].set(x)


    degp = _deg_kernel(col2d, w2d)
    t1, dis = _pre_call(degp, x_pad, W1)
    acc1 = _spmm_kernel(t1, row2d, col2d, w2d)
    t2 = _mid_call(acc1, t1, dis, b1, W2)
    acc2 = _spmm_kernel(t2, row2d, col2d, w2d)
    return _post_call(acc2, t2, dis, b2, Wlin, blin)


# speedup vs baseline: 18.3540x; 1.1250x over previous
import functools

import jax
import jax.numpy as jnp
from jax import lax
from jax.experimental import pallas as pl
from jax.experimental.pallas import tpu as pltpu
from jax.experimental.pallas import tpu_sc as plsc

N_NODES = 10000
D_FEAT = 128
HALF = 64
QW = 32
NC = 2
NS = 16
CHUNK = 128
NPAD = 10240
ROWS_PER_TILE = NPAD // NS
RBLK = 2560
NCHUNKS = 2560
EPAD = NCHUNKS * CHUNK
DEG_CHUNKS_PER_TILE = NCHUNKS // (NC * NS)
SPMM_CHUNKS_PER_TILE = NCHUNKS // NS
KSUP = 1

_mesh = plsc.VectorSubcoreMesh(core_axis_name="c", subcore_axis_name="s")


@functools.partial(
    pl.kernel,
    out_type=jax.ShapeDtypeStruct((NC, NPAD), jnp.float32),
    mesh=_mesh,
    scratch_types=[
        pltpu.VMEM((DEG_CHUNKS_PER_TILE, CHUNK), jnp.int32),
        pltpu.VMEM((DEG_CHUNKS_PER_TILE, CHUNK), jnp.float32),
        pltpu.VMEM((ROWS_PER_TILE,), jnp.float32),
        pltpu.VMEM_SHARED((NPAD,), jnp.float32),
    ],
    compiler_params=pltpu.CompilerParams(use_tc_tiling_on_sc=False),
)
def _deg_kernel(col2d, w2d, deg_out, col_blk, w_blk, zbuf, deg_sh):
    c = lax.axis_index("c")
    s = lax.axis_index("s")
    base = (c * NS + s) * DEG_CHUNKS_PER_TILE
    pltpu.sync_copy(col2d.at[pl.ds(base, DEG_CHUNKS_PER_TILE)], col_blk)
    pltpu.sync_copy(w2d.at[pl.ds(base, DEG_CHUNKS_PER_TILE)], w_blk)

    def _zero(i, _):
        zbuf[pl.ds(i * 16, 16)] = jnp.zeros((16,), jnp.float32)
        return 0

    lax.fori_loop(0, ROWS_PER_TILE // 16, _zero, 0)
    pltpu.sync_copy(zbuf, deg_sh.at[pl.ds(s * ROWS_PER_TILE, ROWS_PER_TILE)])
    plsc.subcore_barrier()

    def _step(j, _):
        pltpu.sync_copy(w_blk.at[j], deg_sh.at[col_blk.at[j]], add=True)
        return 0

    lax.fori_loop(0, DEG_CHUNKS_PER_TILE, _step, 0)
    plsc.subcore_barrier()
    pltpu.sync_copy(
        deg_sh.at[pl.ds(s * ROWS_PER_TILE, ROWS_PER_TILE)],
        deg_out.at[c, pl.ds(s * ROWS_PER_TILE, ROWS_PER_TILE)],
    )


@functools.partial(
    pl.kernel,
    out_type=jax.ShapeDtypeStruct((4, NPAD, QW), jnp.float32),
    mesh=_mesh,
    scratch_types=[
        pltpu.VMEM((SPMM_CHUNKS_PER_TILE, CHUNK), jnp.int32),
        pltpu.VMEM((SPMM_CHUNKS_PER_TILE, CHUNK), jnp.int32),
        pltpu.VMEM((SPMM_CHUNKS_PER_TILE, CHUNK), jnp.float32),
        pltpu.VMEM((CHUNK, QW), jnp.float32),
        pltpu.VMEM((CHUNK, QW), jnp.float32),
        pltpu.VMEM((CHUNK, QW), jnp.float32),
        pltpu.VMEM((CHUNK, QW), jnp.float32),
        pltpu.VMEM_SHARED((NPAD, QW), jnp.float32),
        pltpu.VMEM_SHARED((NPAD, QW), jnp.float32),
        pltpu.SemaphoreType.DMA,
        pltpu.SemaphoreType.DMA,
        pltpu.SemaphoreType.DMA,
        pltpu.SemaphoreType.DMA,
    ],
    compiler_params=pltpu.CompilerParams(use_tc_tiling_on_sc=False),
)
def _spmm_kernel(t_hbm, row2d, col2d, w2d, acc_out,
                 row_blk, col_blk, w_blk, g0, g1, s0, s1, table_q, acc_q,
                 gsem0, gsem1, ssem0, ssem1):
    c = lax.axis_index("c")
    s = lax.axis_index("s")
    base = s * SPMM_CHUNKS_PER_TILE
    pltpu.sync_copy(row2d.at[pl.ds(base, SPMM_CHUNKS_PER_TILE)], row_blk)
    pltpu.sync_copy(col2d.at[pl.ds(base, SPMM_CHUNKS_PER_TILE)], col_blk)
    pltpu.sync_copy(w2d.at[pl.ds(base, SPMM_CHUNKS_PER_TILE)], w_blk)

    def _scale(j, gb, sb):
        def _scale16(g, _2):
            wv = w_blk[j, pl.ds(g * 16, 16)]
            for e_l in range(16):
                wsc = wv[e_l]
                e = g * 16 + e_l
                for k in range(QW // 16):
                    sl = pl.ds(k * 16, 16)
                    sb[e, sl] = gb[e, sl] * wsc
            return 0

        lax.fori_loop(0, CHUNK // 16, _scale16, 0)

    for fq in range(2):
        qg = c * 2 + fq
        pltpu.sync_copy(
            t_hbm.at[qg, pl.ds(s * ROWS_PER_TILE, ROWS_PER_TILE)],
            table_q.at[pl.ds(s * ROWS_PER_TILE, ROWS_PER_TILE)],
        )

        def _zrow(r, _):
            for k in range(QW // 16):
                s0[r, pl.ds(k * 16, 16)] = jnp.zeros((16,), jnp.float32)
            return 0

        lax.fori_loop(0, CHUNK, _zrow, 0)
        for i in range(ROWS_PER_TILE // CHUNK):
            pltpu.sync_copy(
                s0, acc_q.at[pl.ds(s * ROWS_PER_TILE + i * CHUNK, CHUNK)]
            )
        plsc.subcore_barrier()

        pltpu.async_copy(table_q.at[row_blk.at[0]], g0, gsem0)

        def _step(i, _):
            j = 2 * i
            pltpu.make_async_copy(table_q.at[row_blk.at[j]], g0, gsem0).wait()
            pltpu.async_copy(table_q.at[row_blk.at[j + 1]], g1, gsem1)
            _scale(j, g0, s0)
            pltpu.async_copy(s0, acc_q.at[col_blk.at[j]], ssem0, add=True)
            pltpu.make_async_copy(table_q.at[row_blk.at[j + 1]], g1, gsem1).wait()
            pltpu.async_copy(table_q.at[row_blk.at[j + 2]], g0, gsem0)
            _scale(j + 1, g1, s1)
            pltpu.async_copy(s1, acc_q.at[col_blk.at[j + 1]], ssem1, add=True)
            pltpu.make_async_copy(s0, acc_q.at[col_blk.at[j]], ssem0).wait()
            pltpu.make_async_copy(s1, acc_q.at[col_blk.at[j + 1]], ssem1).wait()
            return 0

        lax.fori_loop(0, SPMM_CHUNKS_PER_TILE // 2 - 1, _step, 0)
        jl = SPMM_CHUNKS_PER_TILE - 2
        pltpu.make_async_copy(table_q.at[row_blk.at[jl]], g0, gsem0).wait()
        pltpu.async_copy(table_q.at[row_blk.at[jl + 1]], g1, gsem1)
        _scale(jl, g0, s0)
        pltpu.async_copy(s0, acc_q.at[col_blk.at[jl]], ssem0, add=True)
        pltpu.make_async_copy(table_q.at[row_blk.at[jl + 1]], g1, gsem1).wait()
        _scale(jl + 1, g1, s1)
        pltpu.async_copy(s1, acc_q.at[col_blk.at[jl + 1]], ssem1, add=True)
        pltpu.make_async_copy(s0, acc_q.at[col_blk.at[jl]], ssem0).wait()
        pltpu.make_async_copy(s1, acc_q.at[col_blk.at[jl + 1]], ssem1).wait()
        plsc.subcore_barrier()
        pltpu.sync_copy(
            acc_q.at[pl.ds(s * ROWS_PER_TILE, ROWS_PER_TILE)],
            acc_out.at[qg, pl.ds(s * ROWS_PER_TILE, ROWS_PER_TILE)],
        )
        plsc.subcore_barrier()


def _pre_body(degp_ref, x_ref, w1_ref, t_ref, dis_ref):
    deg = degp_ref[0] + degp_ref[1] + 1.0
    dis = lax.rsqrt(deg)
    dis_ref[...] = dis
    sl = pl.ds(pl.program_id(0) * RBLK, RBLK)
    disb = lax.rsqrt(degp_ref[0, sl] + degp_ref[1, sl] + 1.0)
    xw = jnp.dot(x_ref[...], w1_ref[...], preferred_element_type=jnp.float32)
    t = xw * disb[:, None]
    for q in range(4):
        t_ref[q] = t[:, q * QW:(q + 1) * QW]


def _mid_body(acc_ref, t1_ref, dis_ref, b1_ref, w2_ref, t2_ref):
    pre = jnp.concatenate(
        [acc_ref[q] + t1_ref[q] for q in range(4)], axis=1
    )
    dis = dis_ref[pl.ds(pl.program_id(0) * RBLK, RBLK)]
    h = jnp.maximum(pre * dis[:, None] + b1_ref[...][None, :], 0.0)
    xw2 = jnp.dot(h, w2_ref[...], preferred_element_type=jnp.float32)
    t2 = xw2 * dis[:, None]
    for q in range(4):
        t2_ref[q] = t2[:, q * QW:(q + 1) * QW]


def _post_body(acc_ref, t2_ref, dis_ref, b2_ref, wlin_ref, blin_ref, out_ref):
    pre = jnp.concatenate(
        [acc_ref[q] + t2_ref[q] for q in range(4)], axis=1
    )
    dis = dis_ref[pl.ds(pl.program_id(0) * RBLK, RBLK)]
    h2 = jnp.maximum(pre * dis[:, None] + b2_ref[...][None, :], 0.0)
    y = jnp.dot(h2, wlin_ref[...], preferred_element_type=jnp.float32)
    out_ref[...] = y + blin_ref[...][None, :]


_pre_call = pl.pallas_call(
    _pre_body,
    grid=(NPAD // RBLK,),
    in_specs=[
        pl.BlockSpec((NC, NPAD), lambda i: (0, 0)),
        pl.BlockSpec((RBLK, D_FEAT), lambda i: (i, 0)),
        pl.BlockSpec((D_FEAT, D_FEAT), lambda i: (0, 0)),
    ],
    out_specs=(
        pl.BlockSpec((4, RBLK, QW), lambda i: (0, i, 0)),
        pl.BlockSpec((NPAD,), lambda i: (0,)),
    ),
    out_shape=(
        jax.ShapeDtypeStruct((4, NPAD, QW), jnp.float32),
        jax.ShapeDtypeStruct((NPAD,), jnp.float32),
    ),
)

_mid_call = pl.pallas_call(
    _mid_body,
    grid=(NPAD // RBLK,),
    in_specs=[
        pl.BlockSpec((4, RBLK, QW), lambda i: (0, i, 0)),
        pl.BlockSpec((4, RBLK, QW), lambda i: (0, i, 0)),
        pl.BlockSpec((NPAD,), lambda i: (0,)),
        pl.BlockSpec((D_FEAT,), lambda i: (0,)),
        pl.BlockSpec((D_FEAT, D_FEAT), lambda i: (0, 0)),
    ],
    out_specs=pl.BlockSpec((4, RBLK, QW), lambda i: (0, i, 0)),
    out_shape=jax.ShapeDtypeStruct((4, NPAD, QW), jnp.float32),
)

_post_call = pl.pallas_call(
    _post_body,
    grid=(NPAD // RBLK,),
    in_specs=[
        pl.BlockSpec((4, RBLK, QW), lambda i: (0, i, 0)),
        pl.BlockSpec((4, RBLK, QW), lambda i: (0, i, 0)),
        pl.BlockSpec((NPAD,), lambda i: (0,)),
        pl.BlockSpec((D_FEAT,), lambda i: (0,)),
        pl.BlockSpec((D_FEAT, 1), lambda i: (0, 0)),
        pl.BlockSpec((1,), lambda i: (0,)),
    ],
    out_specs=pl.BlockSpec((RBLK, 1), lambda i: (i, 0)),
    out_shape=jax.ShapeDtypeStruct((N_NODES, 1), jnp.float32),
)


@jax.jit
def kernel(x, edge_index, edge_weight, W1, b1, W2, b2, Wlin, blin):
    row = edge_index[0].astype(jnp.int32)
    col = edge_index[1].astype(jnp.int32)
    w = edge_weight.astype(jnp.float32)
    e = row.shape[0]
    pad = EPAD - e
    row2d = jnp.concatenate([row, jnp.zeros((pad,), jnp.int32)]).reshape(
        NCHUNKS, CHUNK
    )
    col2d = jnp.concatenate([col, jnp.zeros((pad,), jnp.int32)]).reshape(
        NCHUNKS, CHUNK
    )
    w2d = jnp.concatenate([w, jnp.zeros((pad,), jnp.float32)]).reshape(
        NCHUNKS, CHUNK
    )
    x_pad = jnp.zeros((NPAD, D_FEAT), jnp.float32).at[:N_NODES].set(x)


    degp = _deg_kernel(col2d, w2d)
    t1, dis = _pre_call(degp, x_pad, W1)
    acc1 = _spmm_kernel(t1, row2d, col2d, w2d)
    t2 = _mid_call(acc1, t1, dis, b1, W2)
    acc2 = _spmm_kernel(t2, row2d, col2d, w2d)
    return _post_call(acc2, t2, dis, b2, Wlin, blin)
